# baseline (speedup 1.0000x reference)
import functools

import jax
import jax.numpy as jnp
from jax import lax
from jax.experimental import pallas as pl
from jax.experimental.pallas import tpu as pltpu
from jax.experimental.pallas import tpu_sc as plsc

N = 16384
K = 16
NUM_VOTES = 1024
C = 64

_HIGHEST = lax.Precision.HIGHEST


_KNN_R = 256


def _knn_body(p_ref, pt_ref, o_ref):
    R = p_ref.shape[0]
    p = p_ref[...]
    pt = pt_ref[...]
    xx = jnp.sum(p * p, axis=1, keepdims=True)
    yy = jnp.sum(pt * pt, axis=0, keepdims=True)
    xy = jnp.dot(p, pt, precision=_HIGHEST)
    d = (xx + yy) - 2.0 * xy
    cols = lax.broadcasted_iota(jnp.int32, (R, N), 1)
    inf = jnp.float32(jnp.inf)
    m = jnp.min(d, axis=1, keepdims=True)
    for k in range(K):
        cand = jnp.where(d <= m, cols, jnp.int32(N))
        idx = jnp.min(cand, axis=1, keepdims=True)
        o_ref[:, k:k + 1] = idx
        if k < K - 1:
            d = jnp.where(cols == idx, inf, d)
            m = jnp.min(d, axis=1, keepdims=True)


def _knn(points, points_t):
    grid = N // _KNN_R
    return pl.pallas_call(
        _knn_body,
        grid=(grid,),
        in_specs=[
            pl.BlockSpec((_KNN_R, 3), lambda i: (i, 0)),
            pl.BlockSpec((3, N), lambda i: (0, 0)),
        ],
        out_specs=pl.BlockSpec((_KNN_R, K), lambda i: (i, 0)),
        out_shape=jax.ShapeDtypeStruct((N, K), jnp.int32),
        compiler_params=pltpu.CompilerParams(
            dimension_semantics=("arbitrary",)),
    )(points, points_t)


_PROJ_R = 2048


def _proj_body(f_ref, wq_ref, wk_ref, wv_ref, q_ref, k_ref, v_ref):
    f = f_ref[...]
    q_ref[...] = jnp.dot(f, wq_ref[...], precision=_HIGHEST)
    k_ref[...] = jnp.dot(f, wk_ref[...], precision=_HIGHEST)
    v_ref[...] = jnp.dot(f, wv_ref[...], precision=_HIGHEST)


def _proj(features, Wq, Wk, Wv):
    grid = N // _PROJ_R
    w_spec = pl.BlockSpec((C, C), lambda i: (0, 0))
    blk = pl.BlockSpec((_PROJ_R, C), lambda i: (i, 0))
    return pl.pallas_call(
        _proj_body,
        grid=(grid,),
        in_specs=[blk, w_spec, w_spec, w_spec],
        out_specs=[blk, blk, blk],
        out_shape=[jax.ShapeDtypeStruct((N, C), jnp.float32)] * 3,
        compiler_params=pltpu.CompilerParams(
            dimension_semantics=("arbitrary",)),
    )(features, Wq, Wk, Wv)


def _sc_edge_gather(idx_flat, kf, v, pts_pad):
    info = plsc.get_sparse_core_info()
    nc, ns = info.num_cores, info.num_subcores
    nw = nc * ns
    B = N * K
    b_per_w = B // nw
    CH = 128
    n_ch = b_per_w // CH

    mesh = plsc.VectorSubcoreMesh(core_axis_name="c", subcore_axis_name="s")

    @functools.partial(
        pl.kernel, mesh=mesh,
        compiler_params=pltpu.CompilerParams(use_tc_tiling_on_sc=False),
        out_type=[
            jax.ShapeDtypeStruct((B, C), jnp.float32),
            jax.ShapeDtypeStruct((B, C), jnp.float32),
            jax.ShapeDtypeStruct((B, 16), jnp.float32),
        ],
        scratch_types=[
            pltpu.VMEM((CH,), jnp.int32),
            pltpu.VMEM((CH, C), jnp.float32),
            pltpu.VMEM((CH, C), jnp.float32),
            pltpu.VMEM((CH, 16), jnp.float32),
            pltpu.SemaphoreType.DMA,
        ],
    )
    def gather_k(idx_hbm, kf_hbm, v_hbm, pp_hbm, kf_out, v_out, pp_out,
                 idx_v, kfb, vb, ppb, sem):
        wid = lax.axis_index("s") * nc + lax.axis_index("c")
        base = wid * b_per_w

        def body(t, _):
            off = pl.multiple_of(base + t * CH, CH)
            pltpu.sync_copy(idx_hbm.at[pl.ds(off, CH)], idx_v)
            c1 = pltpu.async_copy(kf_hbm.at[idx_v], kfb, sem)
            c2 = pltpu.async_copy(v_hbm.at[idx_v], vb, sem)
            c3 = pltpu.async_copy(pp_hbm.at[idx_v], ppb, sem)
            c1.wait()
            c2.wait()
            c3.wait()
            pltpu.sync_copy(kfb, kf_out.at[pl.ds(off, CH)])
            pltpu.sync_copy(vb, v_out.at[pl.ds(off, CH)])
            pltpu.sync_copy(ppb, pp_out.at[pl.ds(off, CH)])
            return 0

        lax.fori_loop(0, n_ch, body, 0)

    return gather_k(idx_flat, kf, v, pts_pad)


def _sc_seed_gather(idx, out_features, pts_pad):
    info = plsc.get_sparse_core_info()
    nc, ns = info.num_cores, info.num_subcores
    nw = nc * ns
    b_per_w = NUM_VOTES // nw

    mesh = plsc.VectorSubcoreMesh(core_axis_name="c", subcore_axis_name="s")

    @functools.partial(
        pl.kernel, mesh=mesh,
        compiler_params=pltpu.CompilerParams(use_tc_tiling_on_sc=False),
        out_type=[
            jax.ShapeDtypeStruct((NUM_VOTES, C), jnp.float32),
            jax.ShapeDtypeStruct((NUM_VOTES, 16), jnp.float32),
        ],
        scratch_types=[
            pltpu.VMEM((b_per_w,), jnp.int32),
            pltpu.VMEM((b_per_w, C), jnp.float32),
            pltpu.VMEM((b_per_w, 16), jnp.float32),
            pltpu.SemaphoreType.DMA,
        ],
    )
    def gather_k(idx_hbm, f_hbm, pp_hbm, f_out, pp_out, idx_v, fb, ppb, sem):
        wid = lax.axis_index("s") * nc + lax.axis_index("c")
        base = wid * b_per_w
        pltpu.sync_copy(idx_hbm.at[pl.ds(base, b_per_w)], idx_v)
        c1 = pltpu.async_copy(f_hbm.at[idx_v], fb, sem)
        c2 = pltpu.async_copy(pp_hbm.at[idx_v], ppb, sem)
        c1.wait()
        c2.wait()
        pltpu.sync_copy(fb, f_out.at[pl.ds(base, b_per_w)])
        pltpu.sync_copy(ppb, pp_out.at[pl.ds(base, b_per_w)])

    return gather_k(idx, out_features, pts_pad)


_ATT_R = 256


def _attn_body(q_ref, kf_ref, v_ref, pg_ref, p_ref,
               wp1_ref, bp1_ref, wp2_ref, bp2_ref,
               wa1_ref, ba1_ref, wa2_ref, ba2_ref,
               o_ref, h_s, pos_s):
    q = q_ref[...]
    p = p_ref[...]
    wp1 = wp1_ref[...]
    wp2 = wp2_ref[...]
    wa1 = wa1_ref[...]
    wa2 = wa2_ref[...]
    bp1 = bp1_ref[...]
    bp2 = bp2_ref[...]
    ba1 = ba1_ref[...]
    ba2 = ba2_ref[...]
    m = None
    for j in range(K):
        rel = p - pg_ref[j, :, 0:3]
        pos_in = (rel[:, 0:1] * wp1[0:1, :] + rel[:, 1:2] * wp1[1:2, :]
                  + rel[:, 2:3] * wp1[2:3, :]) + bp1
        pos = jnp.dot(jnp.maximum(pos_in, 0.0), wp2, precision=_HIGHEST) + bp2
        pos_s[j] = pos
        x = q - kf_ref[j] + pos
        t = jnp.dot(jnp.maximum(jnp.dot(x, wa1, precision=_HIGHEST) + ba1, 0.0),
                    wa2, precision=_HIGHEST) + ba2
        h_s[j] = t
        m = t if m is None else jnp.maximum(m, t)
    s = None
    acc = None
    for j in range(K):
        e = jnp.exp(h_s[j] - m)
        w = e * (v_ref[j] + pos_s[j])
        s = e if s is None else s + e
        acc = w if acc is None else acc + w
    o_ref[...] = acc / s


def _attn(q, kf_g, v_g, pts_g, points, Wp1, bp1, Wp2, bp2, Wa1, ba1, Wa2, ba2):
    grid = N // _ATT_R
    blk = pl.BlockSpec((_ATT_R, C), lambda i: (i, 0))
    eblk = pl.BlockSpec((K, _ATT_R, C), lambda i: (0, i, 0))
    pblk = pl.BlockSpec((K, _ATT_R, 16), lambda i: (0, i, 0))
    w64 = pl.BlockSpec((C, C), lambda i: (0, 0))
    b64 = pl.BlockSpec((1, C), lambda i: (0, 0))
    return pl.pallas_call(
        _attn_body,
        grid=(grid,),
        in_specs=[
            blk,
            eblk,
            eblk,
            pblk,
            pl.BlockSpec((_ATT_R, 3), lambda i: (i, 0)),
            pl.BlockSpec((3, C), lambda i: (0, 0)), b64,
            w64, b64,
            w64, b64,
            w64, b64,
        ],
        out_specs=blk,
        out_shape=jax.ShapeDtypeStruct((N, C), jnp.float32),
        scratch_shapes=[
            pltpu.VMEM((K, _ATT_R, C), jnp.float32),
            pltpu.VMEM((K, _ATT_R, C), jnp.float32),
        ],
        compiler_params=pltpu.CompilerParams(
            dimension_semantics=("arbitrary",)),
    )(q, kf_g, v_g, pts_g, points, Wp1, bp1, Wp2, bp2, Wa1, ba1, Wa2, ba2)


def _fps_body(px_ref, py_ref, pz_ref, o_ref):
    px = px_ref[...]
    py = py_ref[...]
    pz = pz_ref[...]
    iota = (lax.broadcasted_iota(jnp.int32, (128, 128), 0) * 128
            + lax.broadcasted_iota(jnp.int32, (128, 128), 1))
    oiota = (lax.broadcasted_iota(jnp.int32, (8, 128), 0) * 128
             + lax.broadcasted_iota(jnp.int32, (8, 128), 1))
    inf = jnp.full((128, 128), jnp.inf, jnp.float32)

    def body(i, carry):
        dists, o_acc, lx, ly, lz = carry
        dx = px - lx
        dy = py - ly
        dz = pz - lz
        d = dx * dx + dy * dy + dz * dz
        dn = jnp.minimum(dists, d)
        mx = jnp.max(dn)
        nxt = jnp.min(jnp.where(dn == mx, iota, jnp.int32(2 ** 30)))
        sel = iota == nxt
        zero = jnp.float32(0.0)
        nlx = jnp.sum(jnp.where(sel, px, zero))
        nly = jnp.sum(jnp.where(sel, py, zero))
        nlz = jnp.sum(jnp.where(sel, pz, zero))
        o_acc = jnp.where(oiota == i, nxt, o_acc)
        return (dn, o_acc, nlx, nly, nlz)

    init = (inf, jnp.zeros((8, 128), jnp.int32),
            px[0, 0], py[0, 0], pz[0, 0])
    _, o_acc, _, _, _ = lax.fori_loop(1, NUM_VOTES, body, init)
    o_ref[...] = o_acc


def _fps(px, py, pz):
    full = pl.BlockSpec((128, 128), lambda: (0, 0))
    return pl.pallas_call(
        _fps_body,
        in_specs=[full, full, full],
        out_specs=pl.BlockSpec((8, 128), lambda: (0, 0)),
        out_shape=jax.ShapeDtypeStruct((8, 128), jnp.int32),
    )(px, py, pz)


def _final_body(sf_ref, sp_ref, w1_ref, g_ref, b_ref, w2a_ref, b2a_ref,
                w2b_ref, b2b_ref, vp_ref, vf_ref):
    sf = sf_ref[...]
    h = jnp.dot(sf, w1_ref[...], precision=_HIGHEST)
    mu = jnp.mean(h, axis=0, keepdims=True)
    var = jnp.mean((h - mu) ** 2, axis=0, keepdims=True)
    hn = (h - mu) / jnp.sqrt(var + 1e-5) * g_ref[...] + b_ref[...]
    hr = jnp.maximum(hn, 0.0)
    res_a = jnp.dot(hr, w2a_ref[...], precision=_HIGHEST) + b2a_ref[...]
    res_b = jnp.dot(hr, w2b_ref[...], precision=_HIGHEST) + b2b_ref[...]
    vp_ref[...] = sp_ref[:, 0:3] + res_a
    vf_ref[...] = sf + res_b


def _final(sf, sp_pad, W1, gamma, beta, W2a, b2a, W2b, b2b):
    full = lambda shape: pl.BlockSpec(shape, lambda: (0, 0))
    return pl.pallas_call(
        _final_body,
        in_specs=[
            full((NUM_VOTES, C)), full((NUM_VOTES, 16)),
            full((C, C)), full((1, C)), full((1, C)),
            full((C, 3)), full((1, 3)), full((C, C)), full((1, C)),
        ],
        out_specs=[full((NUM_VOTES, 3)), full((NUM_VOTES, C))],
        out_shape=[
            jax.ShapeDtypeStruct((NUM_VOTES, 3), jnp.float32),
            jax.ShapeDtypeStruct((NUM_VOTES, C), jnp.float32),
        ],
    )(sf, sp_pad, W1, gamma, beta, W2a, b2a, W2b, b2b)


def kernel(points, features, Wq, Wk, Wv, Wp1, bp1, Wp2, bp2, Wa1, ba1,
           Wa2, ba2, W1, gamma, beta, W2, b2):
    points_t = points.T
    pts_pad = jnp.pad(points, ((0, 0), (0, 13)))

    nidx = _knn(points, points_t)
    idx_flat = nidx.T.reshape(-1)

    q, kf, v = _proj(features, Wq, Wk, Wv)

    kf_g, v_g, pg = _sc_edge_gather(idx_flat, kf, v, pts_pad)
    kf_g = kf_g.reshape(K, N, C)
    v_g = v_g.reshape(K, N, C)
    pg = pg.reshape(K, N, 16)

    out_features = _attn(
        q, kf_g, v_g, pg, points,
        Wp1, bp1.reshape(1, C), Wp2, bp2.reshape(1, C),
        Wa1, ba1.reshape(1, C), Wa2, ba2.reshape(1, C))

    px = points[:, 0].reshape(128, 128)
    py = points[:, 1].reshape(128, 128)
    pz = points[:, 2].reshape(128, 128)
    idxs = _fps(px, py, pz).reshape(NUM_VOTES)

    sf, sp_pad = _sc_seed_gather(idxs, out_features, pts_pad)

    vote_points, vote_features = _final(
        sf, sp_pad, W1, gamma.reshape(1, C), beta.reshape(1, C),
        W2[:, 0:3], b2[0:3].reshape(1, 3), W2[:, 3:], b2[3:].reshape(1, C))
    return (vote_points, vote_features)


# speedup vs baseline: 1.1320x; 1.1320x over previous
import functools

import jax
import jax.numpy as jnp
from jax import lax
from jax.experimental import pallas as pl
from jax.experimental.pallas import tpu as pltpu
from jax.experimental.pallas import tpu_sc as plsc

N = 16384
K = 16
NUM_VOTES = 1024
C = 64

_HIGHEST = lax.Precision.HIGHEST


_KNN_R = 256


def _knn_body(p_ref, pt_ref, o_ref):
    R = p_ref.shape[0]
    d = None
    for dim in range(3):
        a = p_ref[:, dim:dim + 1]
        b = pt_ref[dim:dim + 1, :]
        diff = a - b
        sq = diff * diff
        d = sq if d is None else d + sq
    cols = lax.broadcasted_iota(jnp.int32, (R, N), 1)
    inf = jnp.float32(jnp.inf)
    for k in range(K):
        m = jnp.min(d, axis=1, keepdims=True)
        cand = jnp.where(d <= m, cols, jnp.int32(N))
        idx = jnp.min(cand, axis=1, keepdims=True)
        o_ref[:, k:k + 1] = idx
        d = jnp.where(cols == idx, inf, d)


def _knn(points, points_t):
    grid = N // _KNN_R
    return pl.pallas_call(
        _knn_body,
        grid=(grid,),
        in_specs=[
            pl.BlockSpec((_KNN_R, 3), lambda i: (i, 0)),
            pl.BlockSpec((3, N), lambda i: (0, 0)),
        ],
        out_specs=pl.BlockSpec((_KNN_R, K), lambda i: (i, 0)),
        out_shape=jax.ShapeDtypeStruct((N, K), jnp.int32),
        compiler_params=pltpu.CompilerParams(
            dimension_semantics=("arbitrary",)),
    )(points, points_t)


_PROJ_R = 2048


def _proj_body(f_ref, wq_ref, wk_ref, wv_ref, wa1_ref, qa_ref, ka_ref, v_ref):
    f = f_ref[...]
    wa1 = wa1_ref[...]
    q = jnp.dot(f, wq_ref[...], precision=_HIGHEST)
    kf = jnp.dot(f, wk_ref[...], precision=_HIGHEST)
    qa_ref[...] = jnp.dot(q, wa1, precision=_HIGHEST)
    ka_ref[...] = jnp.dot(kf, wa1, precision=_HIGHEST)
    v_ref[...] = jnp.dot(f, wv_ref[...], precision=_HIGHEST)


def _proj(features, Wq, Wk, Wv, Wa1):
    grid = N // _PROJ_R
    w_spec = pl.BlockSpec((C, C), lambda i: (0, 0))
    blk = pl.BlockSpec((_PROJ_R, C), lambda i: (i, 0))
    return pl.pallas_call(
        _proj_body,
        grid=(grid,),
        in_specs=[blk, w_spec, w_spec, w_spec, w_spec],
        out_specs=[blk, blk, blk],
        out_shape=[jax.ShapeDtypeStruct((N, C), jnp.float32)] * 3,
        compiler_params=pltpu.CompilerParams(
            dimension_semantics=("parallel",)),
    )(features, Wq, Wk, Wv, Wa1)


def _sc_edge_gather(idx_flat, kf, v, pts_pad):
    info = plsc.get_sparse_core_info()
    nc, ns = info.num_cores, info.num_subcores
    nw = nc * ns
    B = N * K
    b_per_w = B // nw
    CH = 128
    n_ch = b_per_w // CH

    mesh = plsc.VectorSubcoreMesh(core_axis_name="c", subcore_axis_name="s")

    @functools.partial(
        pl.kernel, mesh=mesh,
        compiler_params=pltpu.CompilerParams(use_tc_tiling_on_sc=False),
        out_type=[
            jax.ShapeDtypeStruct((B, C), jnp.float32),
            jax.ShapeDtypeStruct((B, C), jnp.float32),
            jax.ShapeDtypeStruct((B, 16), jnp.float32),
        ],
        scratch_types=[
            pltpu.VMEM((CH,), jnp.int32),
            pltpu.VMEM((CH, C), jnp.float32),
            pltpu.VMEM((CH, C), jnp.float32),
            pltpu.VMEM((CH, 16), jnp.float32),
            pltpu.SemaphoreType.DMA,
        ],
    )
    def gather_k(idx_hbm, kf_hbm, v_hbm, pp_hbm, kf_out, v_out, pp_out,
                 idx_v, kfb, vb, ppb, sem):
        wid = lax.axis_index("s") * nc + lax.axis_index("c")
        base = wid * b_per_w

        def body(t, _):
            off = pl.multiple_of(base + t * CH, CH)
            pltpu.sync_copy(idx_hbm.at[pl.ds(off, CH)], idx_v)
            c1 = pltpu.async_copy(kf_hbm.at[idx_v], kfb, sem)
            c2 = pltpu.async_copy(v_hbm.at[idx_v], vb, sem)
            c3 = pltpu.async_copy(pp_hbm.at[idx_v], ppb, sem)
            c1.wait()
            c2.wait()
            c3.wait()
            pltpu.sync_copy(kfb, kf_out.at[pl.ds(off, CH)])
            pltpu.sync_copy(vb, v_out.at[pl.ds(off, CH)])
            pltpu.sync_copy(ppb, pp_out.at[pl.ds(off, CH)])
            return 0

        lax.fori_loop(0, n_ch, body, 0)

    return gather_k(idx_flat, kf, v, pts_pad)


def _sc_seed_gather(idx, out_features, pts_pad):
    info = plsc.get_sparse_core_info()
    nc, ns = info.num_cores, info.num_subcores
    nw = nc * ns
    b_per_w = NUM_VOTES // nw

    mesh = plsc.VectorSubcoreMesh(core_axis_name="c", subcore_axis_name="s")

    @functools.partial(
        pl.kernel, mesh=mesh,
        compiler_params=pltpu.CompilerParams(use_tc_tiling_on_sc=False),
        out_type=[
            jax.ShapeDtypeStruct((NUM_VOTES, C), jnp.float32),
            jax.ShapeDtypeStruct((NUM_VOTES, 16), jnp.float32),
        ],
        scratch_types=[
            pltpu.VMEM((b_per_w,), jnp.int32),
            pltpu.VMEM((b_per_w, C), jnp.float32),
            pltpu.VMEM((b_per_w, 16), jnp.float32),
            pltpu.SemaphoreType.DMA,
        ],
    )
    def gather_k(idx_hbm, f_hbm, pp_hbm, f_out, pp_out, idx_v, fb, ppb, sem):
        wid = lax.axis_index("s") * nc + lax.axis_index("c")
        base = wid * b_per_w
        pltpu.sync_copy(idx_hbm.at[pl.ds(base, b_per_w)], idx_v)
        c1 = pltpu.async_copy(f_hbm.at[idx_v], fb, sem)
        c2 = pltpu.async_copy(pp_hbm.at[idx_v], ppb, sem)
        c1.wait()
        c2.wait()
        pltpu.sync_copy(fb, f_out.at[pl.ds(base, b_per_w)])
        pltpu.sync_copy(ppb, pp_out.at[pl.ds(base, b_per_w)])

    return gather_k(idx, out_features, pts_pad)


_ATT_R = 256


def _attn_body(q_ref, kf_ref, v_ref, pg_ref, p_ref,
               wp1_ref, bp1_ref, wp2_ref, bp2_ref,
               wa1_ref, ba1_ref, wa2_ref, ba2_ref,
               o_ref, h_s, pos_s):
    qa = q_ref[...]
    p = p_ref[...]
    wp1 = wp1_ref[...]
    wp2 = wp2_ref[...]
    wa1 = wa1_ref[...]
    wa2 = wa2_ref[...]
    bp1 = bp1_ref[...]
    bp2 = bp2_ref[...]
    ba1 = ba1_ref[...]
    ba2 = ba2_ref[...]
    wcat = jnp.concatenate(
        [wp2, jnp.dot(wp2, wa1, precision=_HIGHEST)], axis=1)
    bcat = jnp.concatenate(
        [bp2, jnp.dot(bp2, wa1, precision=_HIGHEST)], axis=1)
    m = None
    for j in range(K):
        rel = p - pg_ref[j, :, 0:3]
        pos_in = (rel[:, 0:1] * wp1[0:1, :] + rel[:, 1:2] * wp1[1:2, :]
                  + rel[:, 2:3] * wp1[2:3, :]) + bp1
        pp = jnp.dot(jnp.maximum(pos_in, 0.0), wcat,
                     precision=_HIGHEST) + bcat
        pos_s[j] = pp[:, 0:C]
        xa = qa - kf_ref[j] + pp[:, C:2 * C]
        t = jnp.dot(jnp.maximum(xa + ba1, 0.0), wa2,
                    precision=_HIGHEST) + ba2
        h_s[j] = t
        m = t if m is None else jnp.maximum(m, t)
    s = None
    acc = None
    for j in range(K):
        e = jnp.exp(h_s[j] - m)
        w = e * (v_ref[j] + pos_s[j])
        s = e if s is None else s + e
        acc = w if acc is None else acc + w
    o_ref[...] = acc / s


def _attn(q, kf_g, v_g, pts_g, points, Wp1, bp1, Wp2, bp2, Wa1, ba1, Wa2, ba2):
    grid = N // _ATT_R
    blk = pl.BlockSpec((_ATT_R, C), lambda i: (i, 0))
    eblk = pl.BlockSpec((K, _ATT_R, C), lambda i: (0, i, 0))
    pblk = pl.BlockSpec((K, _ATT_R, 16), lambda i: (0, i, 0))
    w64 = pl.BlockSpec((C, C), lambda i: (0, 0))
    b64 = pl.BlockSpec((1, C), lambda i: (0, 0))
    return pl.pallas_call(
        _attn_body,
        grid=(grid,),
        in_specs=[
            blk,
            eblk,
            eblk,
            pblk,
            pl.BlockSpec((_ATT_R, 3), lambda i: (i, 0)),
            pl.BlockSpec((3, C), lambda i: (0, 0)), b64,
            w64, b64,
            w64, b64,
            w64, b64,
        ],
        out_specs=blk,
        out_shape=jax.ShapeDtypeStruct((N, C), jnp.float32),
        scratch_shapes=[
            pltpu.VMEM((K, _ATT_R, C), jnp.float32),
            pltpu.VMEM((K, _ATT_R, C), jnp.float32),
        ],
        compiler_params=pltpu.CompilerParams(
            dimension_semantics=("parallel",)),
    )(q, kf_g, v_g, pts_g, points, Wp1, bp1, Wp2, bp2, Wa1, ba1, Wa2, ba2)


def _fps_body(px_ref, py_ref, pz_ref, pp_ref, o_ref):
    px = px_ref[...]
    py = py_ref[...]
    pz = pz_ref[...]
    iota = (lax.broadcasted_iota(jnp.int32, (128, 128), 0) * 128
            + lax.broadcasted_iota(jnp.int32, (128, 128), 1))
    oiota = (lax.broadcasted_iota(jnp.int32, (8, 128), 0) * 128
             + lax.broadcasted_iota(jnp.int32, (8, 128), 1))
    inf = jnp.full((128, 128), jnp.inf, jnp.float32)

    def body(i, carry):
        dists, o_acc, lx, ly, lz = carry
        dx = px - lx
        dy = py - ly
        dz = pz - lz
        d = dx * dx + dy * dy + dz * dz
        dn = jnp.minimum(dists, d)
        mx = jnp.max(dn)
        nxt = jnp.min(jnp.where(dn == mx, iota, jnp.int32(2 ** 30)))
        row = pp_ref[pl.ds(nxt, 1), :]
        o_acc = jnp.where(oiota == i, nxt, o_acc)
        return (dn, o_acc, row[0, 0], row[0, 1], row[0, 2])

    init = (inf, jnp.zeros((8, 128), jnp.int32),
            px[0, 0], py[0, 0], pz[0, 0])
    _, o_acc, _, _, _ = lax.fori_loop(1, NUM_VOTES, body, init)
    o_ref[...] = o_acc


def _fps(px, py, pz, pts_pad):
    full = pl.BlockSpec((128, 128), lambda: (0, 0))
    return pl.pallas_call(
        _fps_body,
        in_specs=[full, full, full, pl.BlockSpec((N, 16), lambda: (0, 0))],
        out_specs=pl.BlockSpec((8, 128), lambda: (0, 0)),
        out_shape=jax.ShapeDtypeStruct((8, 128), jnp.int32),
    )(px, py, pz, pts_pad)


def _final_body(sf_ref, sp_ref, w1_ref, g_ref, b_ref, w2a_ref, b2a_ref,
                w2b_ref, b2b_ref, vp_ref, vf_ref):
    sf = sf_ref[...]
    h = jnp.dot(sf, w1_ref[...], precision=_HIGHEST)
    mu = jnp.mean(h, axis=0, keepdims=True)
    var = jnp.mean((h - mu) ** 2, axis=0, keepdims=True)
    hn = (h - mu) / jnp.sqrt(var + 1e-5) * g_ref[...] + b_ref[...]
    hr = jnp.maximum(hn, 0.0)
    res_a = jnp.dot(hr, w2a_ref[...], precision=_HIGHEST) + b2a_ref[...]
    res_b = jnp.dot(hr, w2b_ref[...], precision=_HIGHEST) + b2b_ref[...]
    vp_ref[...] = sp_ref[:, 0:3] + res_a
    vf_ref[...] = sf + res_b


def _final(sf, sp_pad, W1, gamma, beta, W2a, b2a, W2b, b2b):
    full = lambda shape: pl.BlockSpec(shape, lambda: (0, 0))
    return pl.pallas_call(
        _final_body,
        in_specs=[
            full((NUM_VOTES, C)), full((NUM_VOTES, 16)),
            full((C, C)), full((1, C)), full((1, C)),
            full((C, 3)), full((1, 3)), full((C, C)), full((1, C)),
        ],
        out_specs=[full((NUM_VOTES, 3)), full((NUM_VOTES, C))],
        out_shape=[
            jax.ShapeDtypeStruct((NUM_VOTES, 3), jnp.float32),
            jax.ShapeDtypeStruct((NUM_VOTES, C), jnp.float32),
        ],
    )(sf, sp_pad, W1, gamma, beta, W2a, b2a, W2b, b2b)


def kernel(points, features, Wq, Wk, Wv, Wp1, bp1, Wp2, bp2, Wa1, ba1,
           Wa2, ba2, W1, gamma, beta, W2, b2):
    points_t = points.T
    pts_pad = jnp.pad(points, ((0, 0), (0, 13)))

    nidx = _knn(points, points_t)
    idx_flat = nidx.T.reshape(-1)

    qa, kfa, v = _proj(features, Wq, Wk, Wv, Wa1)

    kfa_g, v_g, pg = _sc_edge_gather(idx_flat, kfa, v, pts_pad)
    kfa_g = kfa_g.reshape(K, N, C)
    v_g = v_g.reshape(K, N, C)
    pg = pg.reshape(K, N, 16)

    out_features = _attn(
        qa, kfa_g, v_g, pg, points,
        Wp1, bp1.reshape(1, C), Wp2, bp2.reshape(1, C),
        Wa1, ba1.reshape(1, C), Wa2, ba2.reshape(1, C))

    px = points[:, 0].reshape(128, 128)
    py = points[:, 1].reshape(128, 128)
    pz = points[:, 2].reshape(128, 128)
    idxs = _fps(px, py, pz, pts_pad).reshape(NUM_VOTES)

    sf, sp_pad = _sc_seed_gather(idxs, out_features, pts_pad)

    vote_points, vote_features = _final(
        sf, sp_pad, W1, gamma.reshape(1, C), beta.reshape(1, C),
        W2[:, 0:3], b2[0:3].reshape(1, 3), W2[:, 3:], b2[3:].reshape(1, C))
    return (vote_points, vote_features)


# speedup vs baseline: 1.2650x; 1.1175x over previous
import functools

import jax
import jax.numpy as jnp
from jax import lax
from jax.experimental import pallas as pl
from jax.experimental.pallas import tpu as pltpu
from jax.experimental.pallas import tpu_sc as plsc

N = 16384
K = 16
NUM_VOTES = 1024
C = 64

_HIGHEST = lax.Precision.HIGHEST


_KNN_R = 128


def _knn_body(p_ref, pt_ref, o_ref):
    R = p_ref.shape[0]
    W = N // 4
    inf = jnp.float32(jnp.inf)
    gcols = lax.broadcasted_iota(jnp.int32, (R, W), 1)
    vals, idxs = [], []
    for lvl in range(4):
        d = None
        for dim in range(3):
            a = p_ref[:, dim:dim + 1]
            b = pt_ref[dim:dim + 1, lvl * W:(lvl + 1) * W]
            diff = a - b
            sq = diff * diff
            d = sq if d is None else d + sq
        vals.append(d)
        idxs.append(gcols + jnp.int32(lvl * W))

    def cex(i, j):
        av, ac, bv, bc = vals[i], idxs[i], vals[j], idxs[j]
        le = (av < bv) | ((av == bv) & (ac < bc))
        vals[i] = jnp.where(le, av, bv)
        idxs[i] = jnp.where(le, ac, bc)
        vals[j] = jnp.where(le, bv, av)
        idxs[j] = jnp.where(le, bc, ac)

    cex(0, 1)
    cex(2, 3)
    cex(0, 2)
    cex(1, 3)
    cex(1, 2)

    v1, v2, v3, v4 = vals
    i1, i2, i3, i4 = idxs
    for k in range(K):
        m = jnp.min(v1, axis=1, keepdims=True)
        cand = jnp.where(v1 <= m, i1, jnp.int32(N))
        idx = jnp.min(cand, axis=1, keepdims=True)
        o_ref[:, k:k + 1] = idx
        if k < K - 1:
            hit = i1 == idx
            v1 = jnp.where(hit, v2, v1)
            i1 = jnp.where(hit, i2, i1)
            v2 = jnp.where(hit, v3, v2)
            i2 = jnp.where(hit, i3, i2)
            v3 = jnp.where(hit, v4, v3)
            i3 = jnp.where(hit, i4, i3)
            v4 = jnp.where(hit, inf, v4)


def _knn(points, points_t):
    grid = N // _KNN_R
    return pl.pallas_call(
        _knn_body,
        grid=(grid,),
        in_specs=[
            pl.BlockSpec((_KNN_R, 3), lambda i: (i, 0)),
            pl.BlockSpec((3, N), lambda i: (0, 0)),
        ],
        out_specs=pl.BlockSpec((_KNN_R, K), lambda i: (i, 0)),
        out_shape=jax.ShapeDtypeStruct((N, K), jnp.int32),
        compiler_params=pltpu.CompilerParams(
            dimension_semantics=("arbitrary",)),
    )(points, points_t)


_PROJ_R = 2048


def _proj_body(f_ref, wq_ref, wk_ref, wv_ref, wa1_ref, qa_ref, ka_ref, v_ref):
    f = f_ref[...]
    wa1 = wa1_ref[...]
    q = jnp.dot(f, wq_ref[...], precision=_HIGHEST)
    kf = jnp.dot(f, wk_ref[...], precision=_HIGHEST)
    qa_ref[...] = jnp.dot(q, wa1, precision=_HIGHEST)
    ka_ref[...] = jnp.dot(kf, wa1, precision=_HIGHEST)
    v_ref[...] = jnp.dot(f, wv_ref[...], precision=_HIGHEST)


def _proj(features, Wq, Wk, Wv, Wa1):
    grid = N // _PROJ_R
    w_spec = pl.BlockSpec((C, C), lambda i: (0, 0))
    blk = pl.BlockSpec((_PROJ_R, C), lambda i: (i, 0))
    return pl.pallas_call(
        _proj_body,
        grid=(grid,),
        in_specs=[blk, w_spec, w_spec, w_spec, w_spec],
        out_specs=[blk, blk, blk],
        out_shape=[jax.ShapeDtypeStruct((N, C), jnp.float32)] * 3,
        compiler_params=pltpu.CompilerParams(
            dimension_semantics=("parallel",)),
    )(features, Wq, Wk, Wv, Wa1)


def _sc_edge_gather(idx_flat, kf, v, pts_pad):
    info = plsc.get_sparse_core_info()
    nc, ns = info.num_cores, info.num_subcores
    nw = nc * ns
    B = N * K
    b_per_w = B // nw
    CH = 128
    n_ch = b_per_w // CH

    mesh = plsc.VectorSubcoreMesh(core_axis_name="c", subcore_axis_name="s")

    @functools.partial(
        pl.kernel, mesh=mesh,
        compiler_params=pltpu.CompilerParams(use_tc_tiling_on_sc=False),
        out_type=[
            jax.ShapeDtypeStruct((B, C), jnp.float32),
            jax.ShapeDtypeStruct((B, C), jnp.float32),
            jax.ShapeDtypeStruct((B, 16), jnp.float32),
        ],
        scratch_types=[
            pltpu.VMEM((CH,), jnp.int32),
            pltpu.VMEM((CH, C), jnp.float32),
            pltpu.VMEM((CH, C), jnp.float32),
            pltpu.VMEM((CH, 16), jnp.float32),
            pltpu.SemaphoreType.DMA,
        ],
    )
    def gather_k(idx_hbm, kf_hbm, v_hbm, pp_hbm, kf_out, v_out, pp_out,
                 idx_v, kfb, vb, ppb, sem):
        wid = lax.axis_index("s") * nc + lax.axis_index("c")
        base = wid * b_per_w

        def body(t, _):
            off = pl.multiple_of(base + t * CH, CH)
            pltpu.sync_copy(idx_hbm.at[pl.ds(off, CH)], idx_v)
            c1 = pltpu.async_copy(kf_hbm.at[idx_v], kfb, sem)
            c2 = pltpu.async_copy(v_hbm.at[idx_v], vb, sem)
            c3 = pltpu.async_copy(pp_hbm.at[idx_v], ppb, sem)
            c1.wait()
            c2.wait()
            c3.wait()
            pltpu.sync_copy(kfb, kf_out.at[pl.ds(off, CH)])
            pltpu.sync_copy(vb, v_out.at[pl.ds(off, CH)])
            pltpu.sync_copy(ppb, pp_out.at[pl.ds(off, CH)])
            return 0

        lax.fori_loop(0, n_ch, body, 0)

    return gather_k(idx_flat, kf, v, pts_pad)


def _sc_seed_gather(idx, out_features, pts_pad):
    info = plsc.get_sparse_core_info()
    nc, ns = info.num_cores, info.num_subcores
    nw = nc * ns
    b_per_w = NUM_VOTES // nw

    mesh = plsc.VectorSubcoreMesh(core_axis_name="c", subcore_axis_name="s")

    @functools.partial(
        pl.kernel, mesh=mesh,
        compiler_params=pltpu.CompilerParams(use_tc_tiling_on_sc=False),
        out_type=[
            jax.ShapeDtypeStruct((NUM_VOTES, C), jnp.float32),
            jax.ShapeDtypeStruct((NUM_VOTES, 16), jnp.float32),
        ],
        scratch_types=[
            pltpu.VMEM((b_per_w,), jnp.int32),
            pltpu.VMEM((b_per_w, C), jnp.float32),
            pltpu.VMEM((b_per_w, 16), jnp.float32),
            pltpu.SemaphoreType.DMA,
        ],
    )
    def gather_k(idx_hbm, f_hbm, pp_hbm, f_out, pp_out, idx_v, fb, ppb, sem):
        wid = lax.axis_index("s") * nc + lax.axis_index("c")
        base = wid * b_per_w
        pltpu.sync_copy(idx_hbm.at[pl.ds(base, b_per_w)], idx_v)
        c1 = pltpu.async_copy(f_hbm.at[idx_v], fb, sem)
        c2 = pltpu.async_copy(pp_hbm.at[idx_v], ppb, sem)
        c1.wait()
        c2.wait()
        pltpu.sync_copy(fb, f_out.at[pl.ds(base, b_per_w)])
        pltpu.sync_copy(ppb, pp_out.at[pl.ds(base, b_per_w)])

    return gather_k(idx, out_features, pts_pad)


_ATT_R = 256


def _attn_body(q_ref, kf_ref, v_ref, pg_ref, p_ref,
               wp1_ref, bp1_ref, wp2_ref, bp2_ref,
               wa1_ref, ba1_ref, wa2_ref, ba2_ref,
               o_ref, h_s, pos_s):
    qa = q_ref[...]
    p = p_ref[...]
    wp1 = wp1_ref[...]
    wp2 = wp2_ref[...]
    wa1 = wa1_ref[...]
    wa2 = wa2_ref[...]
    bp1 = bp1_ref[...]
    bp2 = bp2_ref[...]
    ba1 = ba1_ref[...]
    ba2 = ba2_ref[...]
    wcat = jnp.concatenate(
        [wp2, jnp.dot(wp2, wa1, precision=_HIGHEST)], axis=1)
    bcat = jnp.concatenate(
        [bp2, jnp.dot(bp2, wa1, precision=_HIGHEST)], axis=1)
    m = None
    for j in range(K):
        rel = p - pg_ref[j, :, 0:3]
        pos_in = (rel[:, 0:1] * wp1[0:1, :] + rel[:, 1:2] * wp1[1:2, :]
                  + rel[:, 2:3] * wp1[2:3, :]) + bp1
        pp = jnp.dot(jnp.maximum(pos_in, 0.0), wcat,
                     precision=_HIGHEST) + bcat
        pos_s[j] = pp[:, 0:C]
        xa = qa - kf_ref[j] + pp[:, C:2 * C]
        t = jnp.dot(jnp.maximum(xa + ba1, 0.0), wa2,
                    precision=_HIGHEST) + ba2
        h_s[j] = t
        m = t if m is None else jnp.maximum(m, t)
    s = None
    acc = None
    for j in range(K):
        e = jnp.exp(h_s[j] - m)
        w = e * (v_ref[j] + pos_s[j])
        s = e if s is None else s + e
        acc = w if acc is None else acc + w
    o_ref[...] = acc / s


def _attn(q, kf_g, v_g, pts_g, points, Wp1, bp1, Wp2, bp2, Wa1, ba1, Wa2, ba2):
    grid = N // _ATT_R
    blk = pl.BlockSpec((_ATT_R, C), lambda i: (i, 0))
    eblk = pl.BlockSpec((K, _ATT_R, C), lambda i: (0, i, 0))
    pblk = pl.BlockSpec((K, _ATT_R, 16), lambda i: (0, i, 0))
    w64 = pl.BlockSpec((C, C), lambda i: (0, 0))
    b64 = pl.BlockSpec((1, C), lambda i: (0, 0))
    return pl.pallas_call(
        _attn_body,
        grid=(grid,),
        in_specs=[
            blk,
            eblk,
            eblk,
            pblk,
            pl.BlockSpec((_ATT_R, 3), lambda i: (i, 0)),
            pl.BlockSpec((3, C), lambda i: (0, 0)), b64,
            w64, b64,
            w64, b64,
            w64, b64,
        ],
        out_specs=blk,
        out_shape=jax.ShapeDtypeStruct((N, C), jnp.float32),
        scratch_shapes=[
            pltpu.VMEM((K, _ATT_R, C), jnp.float32),
            pltpu.VMEM((K, _ATT_R, C), jnp.float32),
        ],
        compiler_params=pltpu.CompilerParams(
            dimension_semantics=("parallel",)),
    )(q, kf_g, v_g, pts_g, points, Wp1, bp1, Wp2, bp2, Wa1, ba1, Wa2, ba2)


def _fps_body(px_ref, py_ref, pz_ref, pp_ref, o_ref):
    px = px_ref[...]
    py = py_ref[...]
    pz = pz_ref[...]
    iota = (lax.broadcasted_iota(jnp.int32, (128, 128), 0) * 128
            + lax.broadcasted_iota(jnp.int32, (128, 128), 1))
    oiota = (lax.broadcasted_iota(jnp.int32, (8, 128), 0) * 128
             + lax.broadcasted_iota(jnp.int32, (8, 128), 1))
    inf = jnp.full((128, 128), jnp.inf, jnp.float32)

    def body(i, carry):
        dists, o_acc, lx, ly, lz = carry
        dx = px - lx
        dy = py - ly
        dz = pz - lz
        d = dx * dx + dy * dy + dz * dz
        dn = jnp.minimum(dists, d)
        mx = jnp.max(dn)
        nxt = jnp.min(jnp.where(dn == mx, iota, jnp.int32(2 ** 30)))
        row = pp_ref[pl.ds(nxt, 1), :]
        o_acc = jnp.where(oiota == i, nxt, o_acc)
        return (dn, o_acc, row[0, 0], row[0, 1], row[0, 2])

    init = (inf, jnp.zeros((8, 128), jnp.int32),
            px[0, 0], py[0, 0], pz[0, 0])
    _, o_acc, _, _, _ = lax.fori_loop(1, NUM_VOTES, body, init)
    o_ref[...] = o_acc


def _fps(px, py, pz, pts_pad):
    full = pl.BlockSpec((128, 128), lambda: (0, 0))
    return pl.pallas_call(
        _fps_body,
        in_specs=[full, full, full, pl.BlockSpec((N, 16), lambda: (0, 0))],
        out_specs=pl.BlockSpec((8, 128), lambda: (0, 0)),
        out_shape=jax.ShapeDtypeStruct((8, 128), jnp.int32),
    )(px, py, pz, pts_pad)


def _final_body(sf_ref, sp_ref, w1_ref, g_ref, b_ref, w2a_ref, b2a_ref,
                w2b_ref, b2b_ref, vp_ref, vf_ref):
    sf = sf_ref[...]
    h = jnp.dot(sf, w1_ref[...], precision=_HIGHEST)
    mu = jnp.mean(h, axis=0, keepdims=True)
    var = jnp.mean((h - mu) ** 2, axis=0, keepdims=True)
    hn = (h - mu) / jnp.sqrt(var + 1e-5) * g_ref[...] + b_ref[...]
    hr = jnp.maximum(hn, 0.0)
    res_a = jnp.dot(hr, w2a_ref[...], precision=_HIGHEST) + b2a_ref[...]
    res_b = jnp.dot(hr, w2b_ref[...], precision=_HIGHEST) + b2b_ref[...]
    vp_ref[...] = sp_ref[:, 0:3] + res_a
    vf_ref[...] = sf + res_b


def _final(sf, sp_pad, W1, gamma, beta, W2a, b2a, W2b, b2b):
    full = lambda shape: pl.BlockSpec(shape, lambda: (0, 0))
    return pl.pallas_call(
        _final_body,
        in_specs=[
            full((NUM_VOTES, C)), full((NUM_VOTES, 16)),
            full((C, C)), full((1, C)), full((1, C)),
            full((C, 3)), full((1, 3)), full((C, C)), full((1, C)),
        ],
        out_specs=[full((NUM_VOTES, 3)), full((NUM_VOTES, C))],
        out_shape=[
            jax.ShapeDtypeStruct((NUM_VOTES, 3), jnp.float32),
            jax.ShapeDtypeStruct((NUM_VOTES, C), jnp.float32),
        ],
    )(sf, sp_pad, W1, gamma, beta, W2a, b2a, W2b, b2b)


def kernel(points, features, Wq, Wk, Wv, Wp1, bp1, Wp2, bp2, Wa1, ba1,
           Wa2, ba2, W1, gamma, beta, W2, b2):
    points_t = points.T
    pts_pad = jnp.pad(points, ((0, 0), (0, 13)))

    nidx = _knn(points, points_t)
    idx_flat = nidx.T.reshape(-1)

    qa, kfa, v = _proj(features, Wq, Wk, Wv, Wa1)

    kfa_g, v_g, pg = _sc_edge_gather(idx_flat, kfa, v, pts_pad)
    kfa_g = kfa_g.reshape(K, N, C)
    v_g = v_g.reshape(K, N, C)
    pg = pg.reshape(K, N, 16)

    out_features = _attn(
        qa, kfa_g, v_g, pg, points,
        Wp1, bp1.reshape(1, C), Wp2, bp2.reshape(1, C),
        Wa1, ba1.reshape(1, C), Wa2, ba2.reshape(1, C))

    px = points[:, 0].reshape(128, 128)
    py = points[:, 1].reshape(128, 128)
    pz = points[:, 2].reshape(128, 128)
    idxs = _fps(px, py, pz, pts_pad).reshape(NUM_VOTES)

    sf, sp_pad = _sc_seed_gather(idxs, out_features, pts_pad)

    vote_points, vote_features = _final(
        sf, sp_pad, W1, gamma.reshape(1, C), beta.reshape(1, C),
        W2[:, 0:3], b2[0:3].reshape(1, 3), W2[:, 3:], b2[3:].reshape(1, C))
    return (vote_points, vote_features)
